# baseline (speedup 1.0000x reference)
import functools

import jax
import jax.numpy as jnp
from jax import lax
from jax.experimental import pallas as pl
from jax.experimental.pallas import tpu as pltpu
from jax.experimental.pallas import tpu_sc as plsc

LANES = 16


@functools.lru_cache(maxsize=None)
def _build(B, S, D, CB):
    info = plsc.get_sparse_core_info()
    NC, NS = info.num_cores, info.num_subcores
    NW = NC * NS
    assert S % NW == 0 and D % LANES == 0 and B % CB == 0
    PW = S // NW
    NCHUNK = B // CB
    ROWS = CB * PW

    mesh = plsc.VectorSubcoreMesh(core_axis_name="c", subcore_axis_name="s")

    @functools.partial(
        pl.kernel,
        mesh=mesh,
        out_type=jax.ShapeDtypeStruct((B * S, D), jnp.float32),
        scratch_types=[
            pltpu.VMEM((ROWS,), jnp.int32),
            pltpu.VMEM((ROWS, D), jnp.float32),
            pltpu.VMEM((PW, D), jnp.float32),
            pltpu.SemaphoreType.DMA,
        ],
    )
    def emb(x_hbm, tab_hbm, pos_hbm, out_hbm, idx_v, rows_v, pos_v, sem):
        wid = lax.axis_index("s") * NC + lax.axis_index("c")
        pbase = pl.multiple_of(wid * PW, PW)
        pltpu.sync_copy(pos_hbm.at[pl.ds(pbase, PW), :], pos_v)

        def chunk(g, carry):
            b0 = g * CB
            for b in range(CB):
                src = pl.multiple_of((b0 + b) * S + pbase, PW)
                pltpu.sync_copy(x_hbm.at[pl.ds(src, PW)],
                                idx_v.at[pl.ds(b * PW, PW)])
            pltpu.async_copy(tab_hbm.at[idx_v], rows_v, sem).wait()

            def add_row(i, c):
                for k in range(D // LANES):
                    pv = pos_v[i, pl.ds(k * LANES, LANES)]
                    for b in range(CB):
                        plsc.addupdate(
                            rows_v.at[b * PW + i, pl.ds(k * LANES, LANES)], pv)
                return c

            lax.fori_loop(0, PW, add_row, 0)

            for b in range(CB):
                dst = pl.multiple_of((b0 + b) * S + pbase, PW)
                pltpu.sync_copy(rows_v.at[pl.ds(b * PW, PW), :],
                                out_hbm.at[pl.ds(dst, PW), :])
            return carry

        lax.fori_loop(0, NCHUNK, chunk, 0)

    return emb


def kernel(x, token_embed, pos_embed):
    B, S = x.shape
    D = token_embed.shape[1]
    xf = x.reshape(B * S).astype(jnp.int32)
    pos = pos_embed[0, :S, :]
    out = _build(B, S, D, 8)(xf, token_embed, pos)
    return out.reshape(B, S, D)


# speedup vs baseline: 1.2435x; 1.2435x over previous
import functools

import jax
import jax.numpy as jnp
from jax import lax
from jax.experimental import pallas as pl
from jax.experimental.pallas import tpu as pltpu
from jax.experimental.pallas import tpu_sc as plsc

LANES = 16


@functools.lru_cache(maxsize=None)
def _build(B, S, D, CB):
    info = plsc.get_sparse_core_info()
    NC, NS = info.num_cores, info.num_subcores
    NW = NC * NS
    assert S % NW == 0 and D % LANES == 0 and B % CB == 0
    PW = S // NW
    NCHUNK = B // CB

    mesh = plsc.VectorSubcoreMesh(core_axis_name="c", subcore_axis_name="s")

    @functools.partial(
        pl.kernel,
        mesh=mesh,
        out_type=jax.ShapeDtypeStruct((B, S, D), jnp.float32),
        scratch_types=[
            pltpu.VMEM((CB * PW,), jnp.int32),
            pltpu.VMEM((CB * PW,), jnp.int32),
            pltpu.VMEM((CB, PW, D), jnp.float32),
            pltpu.VMEM((CB, PW, D), jnp.float32),
            pltpu.VMEM((PW, D), jnp.float32),
            pltpu.SemaphoreType.DMA,
            pltpu.SemaphoreType.DMA,
            pltpu.SemaphoreType.DMA,
            pltpu.SemaphoreType.DMA,
        ],
    )
    def emb(x_hbm, tab_hbm, pos_hbm, out_hbm,
            idx0, idx1, rows0, rows1, pos_v, g0, g1, w0, w1):
        wid = lax.axis_index("s") * NC + lax.axis_index("c")
        pbase = pl.multiple_of(wid * PW, PW)
        idx = (idx0, idx1)
        rows = (rows0, rows1)
        gsem = (g0, g1)
        wsem = (w0, w1)

        pltpu.sync_copy(pos_hbm.at[pl.ds(pbase, PW), :], pos_v)

        def stage(g, buf):
            for b in range(CB):
                src = pl.multiple_of((g * CB + b) * S + pbase, PW)
                pltpu.sync_copy(x_hbm.at[pl.ds(src, PW)],
                                idx[buf].at[pl.ds(b * PW, PW)])
            return [
                pltpu.async_copy(tab_hbm.at[idx[buf].at[pl.ds(b * PW, PW)]],
                                 rows[buf].at[b], gsem[buf])
                for b in range(CB)
            ]

        def add_pos(buf):
            def add_row(i, c):
                for k in range(D // LANES):
                    pv = pos_v[i, pl.ds(k * LANES, LANES)]
                    for b in range(CB):
                        plsc.addupdate(
                            rows[buf].at[b, i, pl.ds(k * LANES, LANES)], pv)
                return c

            lax.fori_loop(0, PW, add_row, 0)

        gh = [None, None]
        wh = [None, None]
        gh[0] = stage(0, 0)
        for g in range(NCHUNK):
            cur = g & 1
            nxt = cur ^ 1
            if g + 1 < NCHUNK:
                if wh[nxt] is not None:
                    wh[nxt].wait()
                    wh[nxt] = None
                gh[nxt] = stage(g + 1, nxt)
            for h in gh[cur]:
                h.wait()
            add_pos(cur)
            wh[cur] = pltpu.async_copy(
                rows[cur],
                out_hbm.at[pl.ds(g * CB, CB), pl.ds(pbase, PW), :],
                wsem[cur])
        for buf in range(2):
            if wh[buf] is not None:
                wh[buf].wait()

    return emb


def kernel(x, token_embed, pos_embed):
    B, S = x.shape
    D = token_embed.shape[1]
    xf = x.reshape(B * S).astype(jnp.int32)
    pos = pos_embed[0, :S, :]
    return _build(B, S, D, 4)(xf, token_embed, pos)


# speedup vs baseline: 1.5954x; 1.2829x over previous
import functools

import jax
import jax.numpy as jnp
from jax import lax
from jax.experimental import pallas as pl
from jax.experimental.pallas import tpu as pltpu
from jax.experimental.pallas import tpu_sc as plsc

LANES = 16


@functools.lru_cache(maxsize=None)
def _build(B, S, D, CB):
    info = plsc.get_sparse_core_info()
    NC, NS = info.num_cores, info.num_subcores
    NW = NC * NS
    assert S % NW == 0 and D % LANES == 0 and B % CB == 0
    PW = S // NW
    NCHUNK = B // CB
    ROWS = CB * PW

    mesh = plsc.VectorSubcoreMesh(core_axis_name="c", subcore_axis_name="s")

    @functools.partial(
        pl.kernel,
        mesh=mesh,
        out_type=jax.ShapeDtypeStruct((B, S, D), jnp.float32),
        scratch_types=[
            pltpu.VMEM((B * PW,), jnp.int32),
            pltpu.VMEM((ROWS, D), jnp.float32),
            pltpu.VMEM((ROWS, D), jnp.float32),
            pltpu.VMEM((PW, D), jnp.float32),
            pltpu.SemaphoreType.DMA,
            pltpu.SemaphoreType.DMA,
            pltpu.SemaphoreType.DMA,
            pltpu.SemaphoreType.DMA,
            pltpu.SemaphoreType.DMA,
        ],
    )
    def emb(x_hbm, tab_hbm, pos_hbm, out_hbm,
            idx_all, rows0, rows1, pos_v, isem, g0, g1, w0, w1):
        wid = lax.axis_index("s") * NC + lax.axis_index("c")
        pbase = pl.multiple_of(wid * PW, PW)
        rows = (rows0, rows1)
        gsem = (g0, g1)
        wsem = (w0, w1)

        for wave in range(0, B, 16):
            hs = []
            for b in range(wave, wave + 16):
                src = pl.multiple_of(b * S + pbase, PW)
                hs.append(pltpu.async_copy(
                    x_hbm.at[pl.ds(src, PW)],
                    idx_all.at[pl.ds(b * PW, PW)], isem))
            for h in hs:
                h.wait()
        pltpu.sync_copy(pos_hbm.at[pl.ds(pbase, PW), :], pos_v)

        def stage(g, buf):
            off = pl.multiple_of(g * ROWS, ROWS)
            return pltpu.async_copy(
                tab_hbm.at[idx_all.at[pl.ds(off, ROWS)]], rows[buf], gsem[buf])

        def add_pos(buf):
            def add_row(i, c):
                for k in range(D // LANES):
                    pv = pos_v[i, pl.ds(k * LANES, LANES)]
                    for b in range(CB):
                        plsc.addupdate(
                            rows[buf].at[b * PW + i, pl.ds(k * LANES, LANES)],
                            pv)
                return c

            lax.fori_loop(0, PW, add_row, 0)

        gh = [None, None]
        wh = [[], []]
        gh[0] = stage(0, 0)
        for g in range(NCHUNK):
            cur = g & 1
            nxt = cur ^ 1
            if g + 1 < NCHUNK:
                for h in wh[nxt]:
                    h.wait()
                wh[nxt] = []
                gh[nxt] = stage(g + 1, nxt)
            gh[cur].wait()
            add_pos(cur)
            wh[cur] = [
                pltpu.async_copy(
                    rows[cur].at[pl.ds(b * PW, PW), :],
                    out_hbm.at[g * CB + b, pl.ds(pbase, PW), :],
                    wsem[cur])
                for b in range(CB)
            ]
        for buf in range(2):
            for h in wh[buf]:
                h.wait()

    return emb


def kernel(x, token_embed, pos_embed):
    B, S = x.shape
    D = token_embed.shape[1]
    xf = x.reshape(B * S).astype(jnp.int32)
    pos = pos_embed.reshape(-1, D)
    return _build(B, S, D, 4)(xf, token_embed, pos)


# speedup vs baseline: 1.6312x; 1.0224x over previous
import functools

import jax
import jax.numpy as jnp
from jax import lax
from jax.experimental import pallas as pl
from jax.experimental.pallas import tpu as pltpu
from jax.experimental.pallas import tpu_sc as plsc

LANES = 16


@functools.lru_cache(maxsize=None)
def _build(B, S, D, CB):
    info = plsc.get_sparse_core_info()
    NC, NS = info.num_cores, info.num_subcores
    NW = NC * NS
    assert S % NW == 0 and D % LANES == 0 and B % CB == 0
    PW = S // NW
    NCHUNK = B // CB
    ROWS = CB * PW

    mesh = plsc.VectorSubcoreMesh(core_axis_name="c", subcore_axis_name="s")

    @functools.partial(
        pl.kernel,
        mesh=mesh,
        out_type=jax.ShapeDtypeStruct((B, S, D), jnp.float32),
        scratch_types=[
            pltpu.VMEM((B * PW,), jnp.int32),
            pltpu.VMEM((ROWS, D), jnp.float32),
            pltpu.VMEM((ROWS, D), jnp.float32),
            pltpu.VMEM((ROWS, D), jnp.float32),
            pltpu.VMEM((PW, D), jnp.float32),
            pltpu.SemaphoreType.DMA,
            pltpu.SemaphoreType.DMA,
            pltpu.SemaphoreType.DMA,
            pltpu.SemaphoreType.DMA,
            pltpu.SemaphoreType.DMA,
            pltpu.SemaphoreType.DMA,
            pltpu.SemaphoreType.DMA,
        ],
    )
    def emb(x_hbm, tab_hbm, pos_hbm, out_hbm,
            idx_all, rows0, rows1, rows2, pos_v,
            isem, g0, g1, g2, w0, w1, w2):
        wid = lax.axis_index("s") * NC + lax.axis_index("c")
        pbase = pl.multiple_of(wid * PW, PW)
        rows = (rows0, rows1, rows2)
        gsem = (g0, g1, g2)
        wsem = (w0, w1, w2)

        for wave in range(0, B, 16):
            hs = []
            for b in range(wave, wave + 16):
                src = pl.multiple_of(b * S + pbase, PW)
                hs.append(pltpu.async_copy(
                    x_hbm.at[pl.ds(src, PW)],
                    idx_all.at[pl.ds(b * PW, PW)], isem))
            for h in hs:
                h.wait()
        pltpu.sync_copy(pos_hbm.at[pl.ds(pbase, PW), :], pos_v)

        def stage(g, buf):
            off = pl.multiple_of(g * ROWS, ROWS)
            return pltpu.async_copy(
                tab_hbm.at[idx_all.at[pl.ds(off, ROWS)]], rows[buf], gsem[buf])

        def add_pos(buf):
            def add_row(i, c):
                for k in range(D // LANES):
                    pv = pos_v[i, pl.ds(k * LANES, LANES)]
                    for b in range(CB):
                        plsc.addupdate(
                            rows[buf].at[b * PW + i, pl.ds(k * LANES, LANES)],
                            pv)
                return c

            lax.fori_loop(0, PW, add_row, 0)

        NBUF = 3
        gh = [None] * NBUF
        wh = [[] for _ in range(NBUF)]
        gh[0] = stage(0, 0)
        gh[1] = stage(1, 1)
        for g in range(NCHUNK):
            cur = g % NBUF
            if g + 2 < NCHUNK:
                nb = (g + 2) % NBUF
                for h in wh[nb]:
                    h.wait()
                wh[nb] = []
                gh[nb] = stage(g + 2, nb)
            gh[cur].wait()
            add_pos(cur)
            wh[cur] = [
                pltpu.async_copy(
                    rows[cur].at[pl.ds(b * PW, PW), :],
                    out_hbm.at[g * CB + b, pl.ds(pbase, PW), :],
                    wsem[cur])
                for b in range(CB)
            ]
        for buf in range(NBUF):
            for h in wh[buf]:
                h.wait()

    return emb


def kernel(x, token_embed, pos_embed):
    B, S = x.shape
    D = token_embed.shape[1]
    xf = x.reshape(B * S).astype(jnp.int32)
    pos = pos_embed.reshape(-1, D)
    return _build(B, S, D, 4)(xf, token_embed, pos)
